# baseline (speedup 1.0000x reference)
import functools

import jax
import jax.numpy as jnp
from jax import lax
from jax.experimental import pallas as pl
from jax.experimental.pallas import tpu as pltpu
from jax.experimental.pallas import tpu_sc as plsc

_B, _L = 1024, 200
_D = 112
_N = _B * _L
_INFO = plsc.get_sparse_core_info()
_NC, _NS = _INFO.num_cores, _INFO.num_subcores
_NW = _NC * _NS
_PER_W = _N // _NW
_C = 128
_NCHUNK = _PER_W // _C

_mesh = plsc.VectorSubcoreMesh(core_axis_name="c", subcore_axis_name="s")


@functools.partial(
    pl.kernel,
    mesh=_mesh,
    out_type=jax.ShapeDtypeStruct((_N, _D), jnp.float32),
    scratch_types=[
        pltpu.VMEM((_C,), jnp.int32),
        pltpu.VMEM((_C, _D), jnp.float32),
        pltpu.SemaphoreType.DMA,
    ],
    compiler_params=pltpu.CompilerParams(use_tc_tiling_on_sc=False),
)
def _gather1(words, wt, out, iw, bw, sem):
    wid = lax.axis_index("s") * _NC + lax.axis_index("c")
    wbase = wid * _PER_W

    def body(k, carry):
        base = wbase + k * _C
        pltpu.sync_copy(words.at[pl.ds(base, _C)], iw)
        pltpu.async_copy(wt.at[iw], bw, sem).wait()
        pltpu.sync_copy(bw, out.at[pl.ds(base, _C)])
        return carry

    lax.fori_loop(0, _NCHUNK, body, 0)


def kernel(words, tags, lemmas, word_table, tag_table, lemma_table):
    wt112 = jnp.pad(word_table, ((0, 0), (0, 12)))
    word_embed = _gather1(words.reshape(-1), wt112)[:, :100].reshape(_B, _L, 100)
    tag_embed = jnp.take(tag_table, tags, axis=0)
    lemma_embed = jnp.take(lemma_table, lemmas, axis=0)
    return jnp.concatenate([word_embed, tag_embed, lemma_embed], axis=-1)


# speedup vs baseline: 1.5872x; 1.5872x over previous
import functools

import jax
import jax.numpy as jnp
from jax import lax
from jax.experimental import pallas as pl
from jax.experimental.pallas import tpu as pltpu
from jax.experimental.pallas import tpu_sc as plsc

_B, _L = 1024, 200
_D = 100
_DP = 128
_N = _B * _L
_INFO = plsc.get_sparse_core_info()
_NC, _NS = _INFO.num_cores, _INFO.num_subcores
_NW = _NC * _NS
_PER_W = _N // _NW
_G = 128
_C = 256
_NSUB = _C // _G
_NCHUNK = _PER_W // _C

_mesh = plsc.VectorSubcoreMesh(core_axis_name="c", subcore_axis_name="s")


@functools.partial(
    pl.kernel,
    mesh=_mesh,
    out_type=(
        jax.ShapeDtypeStruct((_N, _DP), jnp.float32),
        jax.ShapeDtypeStruct((_N, _DP), jnp.float32),
        jax.ShapeDtypeStruct((_N, _DP), jnp.float32),
    ),
    scratch_types=[
        pltpu.VMEM((_C,), jnp.int32),
        pltpu.VMEM((_C,), jnp.int32),
        pltpu.VMEM((_C,), jnp.int32),
        pltpu.VMEM((_C, _DP), jnp.float32),
        pltpu.VMEM((_C, _DP), jnp.float32),
        pltpu.VMEM((_C, _DP), jnp.float32),
        pltpu.SemaphoreType.DMA,
    ],
)
def _gather3(words, tags, lemmas, wt, tt, lt, ow, ot, ol,
             iw, it, il, bw, bt, bl, sem):
    wid = lax.axis_index("s") * _NC + lax.axis_index("c")
    wbase = wid * _PER_W

    def body(k, carry):
        base = wbase + k * _C
        pltpu.sync_copy(words.at[pl.ds(base, _C)], iw)
        pltpu.sync_copy(tags.at[pl.ds(base, _C)], it)
        pltpu.sync_copy(lemmas.at[pl.ds(base, _C)], il)
        copies = []
        for tab, idx, buf in ((wt, iw, bw), (tt, it, bt), (lt, il, bl)):
            for j in range(_NSUB):
                s = pl.ds(j * _G, _G)
                copies.append(
                    pltpu.async_copy(tab.at[idx.at[s]], buf.at[s], sem))
        for c in copies:
            c.wait()
        pltpu.sync_copy(bw, ow.at[pl.ds(base, _C)])
        pltpu.sync_copy(bt, ot.at[pl.ds(base, _C)])
        pltpu.sync_copy(bl, ol.at[pl.ds(base, _C)])
        return carry

    lax.fori_loop(0, _NCHUNK, body, 0)


def kernel(words, tags, lemmas, word_table, tag_table, lemma_table):
    pad = ((0, 0), (0, _DP - _D))
    ow, ot, ol = _gather3(
        words.reshape(-1), tags.reshape(-1), lemmas.reshape(-1),
        jnp.pad(word_table, pad), jnp.pad(tag_table, pad),
        jnp.pad(lemma_table, pad),
    )
    embed = jnp.concatenate([ow[:, :_D], ot[:, :_D], ol[:, :_D]], axis=-1)
    return embed.reshape(_B, _L, 3 * _D)


# speedup vs baseline: 2.8898x; 1.8207x over previous
import functools

import jax
import jax.numpy as jnp
from jax import lax
from jax.experimental import pallas as pl
from jax.experimental.pallas import tpu as pltpu
from jax.experimental.pallas import tpu_sc as plsc

_B, _L = 1024, 200
_D = 100
_DP = 128
_N = _B * _L
_INFO = plsc.get_sparse_core_info()
_NC, _NS = _INFO.num_cores, _INFO.num_subcores
_NW = _NC * _NS
_PER_W = _N // _NW
_G = 128
_C = 256
_NSUB = _C // _G
_NCHUNK = _PER_W // _C

_mesh = plsc.VectorSubcoreMesh(core_axis_name="c", subcore_axis_name="s")


@functools.partial(
    pl.kernel,
    mesh=_mesh,
    out_type=(
        jax.ShapeDtypeStruct((_N, _DP), jnp.float32),
        jax.ShapeDtypeStruct((_N, _DP), jnp.float32),
        jax.ShapeDtypeStruct((_N, _DP), jnp.float32),
    ),
    scratch_types=[
        pltpu.VMEM((_C,), jnp.int32),
        pltpu.VMEM((_C,), jnp.int32),
        pltpu.VMEM((_C,), jnp.int32),
        pltpu.VMEM((_C, _DP), jnp.float32),
        pltpu.VMEM((_C, _DP), jnp.float32),
        pltpu.VMEM((_C, _DP), jnp.float32),
        pltpu.SemaphoreType.DMA,
    ],
)
def _gather3(words, tags, lemmas, wt, tt, lt, ow, ot, ol,
             iw, it, il, bw, bt, bl, sem):
    wid = lax.axis_index("s") * _NC + lax.axis_index("c")
    wbase = wid * _PER_W

    def body(k, carry):
        base = wbase + k * _C
        pltpu.sync_copy(words.at[pl.ds(base, _C)], iw)
        pltpu.sync_copy(tags.at[pl.ds(base, _C)], it)
        pltpu.sync_copy(lemmas.at[pl.ds(base, _C)], il)
        copies = []
        for tab, idx, buf in ((wt, iw, bw), (tt, it, bt), (lt, il, bl)):
            for j in range(_NSUB):
                s = pl.ds(j * _G, _G)
                copies.append(
                    pltpu.async_copy(tab.at[idx.at[s]], buf.at[s], sem))
        for c in copies:
            c.wait()
        pltpu.sync_copy(bw, ow.at[pl.ds(base, _C)])
        pltpu.sync_copy(bt, ot.at[pl.ds(base, _C)])
        pltpu.sync_copy(bl, ol.at[pl.ds(base, _C)])
        return carry

    lax.fori_loop(0, _NCHUNK, body, 0)


def _pad_body(t_ref, o_ref):
    x = t_ref[...]
    z = jnp.zeros((x.shape[0], _DP - _D), x.dtype)
    o_ref[...] = jnp.concatenate([x, z], axis=1)


def _pad128(table, rows_per_block):
    v = table.shape[0]
    grid = v // rows_per_block
    return pl.pallas_call(
        _pad_body,
        grid=(grid,),
        in_specs=[pl.BlockSpec((rows_per_block, _D), lambda i: (i, 0))],
        out_specs=pl.BlockSpec((rows_per_block, _DP), lambda i: (i, 0)),
        out_shape=jax.ShapeDtypeStruct((v, _DP), jnp.float32),
    )(table)


def kernel(words, tags, lemmas, word_table, tag_table, lemma_table):
    ow, ot, ol = _gather3(
        words.reshape(-1), tags.reshape(-1), lemmas.reshape(-1),
        _pad128(word_table, 4000), _pad128(tag_table, 1000),
        _pad128(lemma_table, 4000),
    )
    embed = jnp.concatenate([ow[:, :_D], ot[:, :_D], ol[:, :_D]], axis=-1)
    return embed.reshape(_B, _L, 3 * _D)


# speedup vs baseline: 3.1013x; 1.0732x over previous
import functools

import jax
import jax.numpy as jnp
from jax import lax
from jax.experimental import pallas as pl
from jax.experimental.pallas import tpu as pltpu
from jax.experimental.pallas import tpu_sc as plsc

_B, _L = 1024, 200
_D = 100
_DP = 128
_N = _B * _L
_INFO = plsc.get_sparse_core_info()
_NC, _NS = _INFO.num_cores, _INFO.num_subcores
_NW = _NC * _NS
_PER_W = _N // _NW
_G = 128

_mesh = plsc.VectorSubcoreMesh(core_axis_name="c", subcore_axis_name="s")


def _make_gather(n_tables, chunk):
    nsub = chunk // _G
    nchunk = _PER_W // chunk
    assert chunk % _G == 0 and _PER_W % chunk == 0

    out_type = tuple(
        jax.ShapeDtypeStruct((_N, _DP), jnp.float32) for _ in range(n_tables))
    scratch = (
        [pltpu.VMEM((chunk,), jnp.int32) for _ in range(n_tables)]
        + [pltpu.VMEM((chunk, _DP), jnp.float32) for _ in range(n_tables)]
        + [pltpu.SemaphoreType.DMA]
    )

    @functools.partial(
        pl.kernel, mesh=_mesh, out_type=out_type, scratch_types=scratch)
    def gather(*refs):
        idx_hbm = refs[:n_tables]
        tabs = refs[n_tables:2 * n_tables]
        outs = refs[2 * n_tables:3 * n_tables]
        idx_v = refs[3 * n_tables:4 * n_tables]
        bufs = refs[4 * n_tables:5 * n_tables]
        sem = refs[5 * n_tables]

        wid = lax.axis_index("s") * _NC + lax.axis_index("c")
        wbase = wid * _PER_W

        def body(k, carry):
            base = wbase + k * chunk
            for t in range(n_tables):
                pltpu.sync_copy(idx_hbm[t].at[pl.ds(base, chunk)], idx_v[t])
            copies = []
            for t in range(n_tables):
                for j in range(nsub):
                    s = pl.ds(j * _G, _G)
                    copies.append(pltpu.async_copy(
                        tabs[t].at[idx_v[t].at[s]], bufs[t].at[s], sem))
            for c in copies:
                c.wait()
            for t in range(n_tables):
                pltpu.sync_copy(bufs[t], outs[t].at[pl.ds(base, chunk)])
            return carry

        lax.fori_loop(0, nchunk, body, 0)

    return gather


_gather2 = _make_gather(2, 256)
_gather1w = _make_gather(1, 640)


def _pad_body(t_ref, o_ref):
    x = t_ref[...]
    z = jnp.zeros((x.shape[0], _DP - _D), x.dtype)
    o_ref[...] = jnp.concatenate([x, z], axis=1)


def _pad128(table, rows_per_block):
    v = table.shape[0]
    grid = v // rows_per_block
    return pl.pallas_call(
        _pad_body,
        grid=(grid,),
        in_specs=[pl.BlockSpec((rows_per_block, _D), lambda i: (i, 0))],
        out_specs=pl.BlockSpec((rows_per_block, _DP), lambda i: (i, 0)),
        out_shape=jax.ShapeDtypeStruct((v, _DP), jnp.float32),
    )(table)


def kernel(words, tags, lemmas, word_table, tag_table, lemma_table):
    ot, ol = _gather2(
        tags.reshape(-1), lemmas.reshape(-1),
        _pad128(tag_table, 1000), _pad128(lemma_table, 5000),
    )
    ow, = _gather1w(words.reshape(-1), _pad128(word_table, 8000))
    embed = jnp.concatenate([ow[:, :_D], ot[:, :_D], ol[:, :_D]], axis=-1)
    return embed.reshape(_B, _L, 3 * _D)


# speedup vs baseline: 4.1101x; 1.3253x over previous
import functools

import jax
import jax.numpy as jnp
from jax import lax
from jax.experimental import pallas as pl
from jax.experimental.pallas import tpu as pltpu
from jax.experimental.pallas import tpu_sc as plsc

_B, _L = 1024, 200
_D = 100
_DP = 128
_N = _B * _L
_INFO = plsc.get_sparse_core_info()
_NC, _NS = _INFO.num_cores, _INFO.num_subcores
_NW = _NC * _NS
_PER_W = _N // _NW
_G = 128

_mesh = plsc.VectorSubcoreMesh(core_axis_name="c", subcore_axis_name="s")


def _make_gather(n_tables, chunk):
    nsub = chunk // _G
    nchunk = _PER_W // chunk
    assert chunk % _G == 0 and _PER_W % chunk == 0

    out_type = tuple(
        jax.ShapeDtypeStruct((_N, _DP), jnp.float32) for _ in range(n_tables))
    scratch = [
        pltpu.VMEM((chunk,), jnp.int32),
        pltpu.VMEM((chunk, _DP), jnp.float32),
        pltpu.SemaphoreType.DMA,
    ]

    @functools.partial(
        pl.kernel, mesh=_mesh, out_type=out_type, scratch_types=scratch)
    def gather(*refs):
        idx_hbm = refs[:n_tables]
        tabs = refs[n_tables:2 * n_tables]
        outs = refs[2 * n_tables:3 * n_tables]
        idx_v, buf, sem = refs[3 * n_tables:]

        wid = lax.axis_index("s") * _NC + lax.axis_index("c")
        wbase = wid * _PER_W

        for t in range(n_tables):
            def body(k, carry, t=t):
                base = wbase + k * chunk
                pltpu.sync_copy(idx_hbm[t].at[pl.ds(base, chunk)], idx_v)
                copies = []
                for j in range(nsub):
                    s = pl.ds(j * _G, _G)
                    copies.append(pltpu.async_copy(
                        tabs[t].at[idx_v.at[s]], buf.at[s], sem))
                for c in copies:
                    c.wait()
                pltpu.sync_copy(buf, outs[t].at[pl.ds(base, chunk)])
                return carry

            lax.fori_loop(0, nchunk, body, 0)

    return gather


_gather2 = _make_gather(2, 640)
_gather1w = _make_gather(1, 640)


def _pad128_mxu(table):
    eye = jnp.eye(_D, _DP, dtype=jnp.float32)
    return lax.dot_general(
        table, eye, (((1,), (0,)), ((), ())),
        precision=lax.Precision.HIGHEST,
        preferred_element_type=jnp.float32,
    )


def kernel(words, tags, lemmas, word_table, tag_table, lemma_table):
    ot, ol = _gather2(
        tags.reshape(-1), lemmas.reshape(-1),
        _pad128_mxu(tag_table), _pad128_mxu(lemma_table),
    )
    ow, = _gather1w(words.reshape(-1), _pad128_mxu(word_table))
    embed = jnp.concatenate([ow[:, :_D], ot[:, :_D], ol[:, :_D]], axis=-1)
    return embed.reshape(_B, _L, 3 * _D)


# speedup vs baseline: 4.1212x; 1.0027x over previous
import functools

import jax
import jax.numpy as jnp
from jax import lax
from jax.experimental import pallas as pl
from jax.experimental.pallas import tpu as pltpu
from jax.experimental.pallas import tpu_sc as plsc

_B, _L = 1024, 200
_D = 100
_DP = 128
_N = _B * _L
_INFO = plsc.get_sparse_core_info()
_NC, _NS = _INFO.num_cores, _INFO.num_subcores
_NW = _NC * _NS
_PER_W = _N // _NW
_G = 128

_mesh = plsc.VectorSubcoreMesh(core_axis_name="c", subcore_axis_name="s")


def _make_gather(n_tables, chunk):
    nsub = chunk // _G
    nchunk = _PER_W // chunk
    assert chunk % _G == 0 and _PER_W % chunk == 0

    out_type = tuple(
        jax.ShapeDtypeStruct((_N, _DP), jnp.float32) for _ in range(n_tables))
    scratch = [
        pltpu.VMEM((_PER_W,), jnp.int32),
        pltpu.VMEM((chunk, _DP), jnp.float32),
        pltpu.SemaphoreType.DMA,
    ]

    @functools.partial(
        pl.kernel, mesh=_mesh, out_type=out_type, scratch_types=scratch)
    def gather(*refs):
        idx_hbm = refs[:n_tables]
        tabs = refs[n_tables:2 * n_tables]
        outs = refs[2 * n_tables:3 * n_tables]
        idx_v, buf, sem = refs[3 * n_tables:]

        wid = lax.axis_index("s") * _NC + lax.axis_index("c")
        wbase = wid * _PER_W

        for t in range(n_tables):
            pltpu.sync_copy(idx_hbm[t].at[pl.ds(wbase, _PER_W)], idx_v)

            def body(k, carry, t=t):
                base = wbase + k * chunk
                copies = []
                for j in range(nsub):
                    s = pl.ds(j * _G, _G)
                    copies.append(pltpu.async_copy(
                        tabs[t].at[idx_v.at[pl.ds(k * chunk + j * _G, _G)]],
                        buf.at[s], sem))
                for c in copies:
                    c.wait()
                pltpu.sync_copy(buf, outs[t].at[pl.ds(base, chunk)])
                return carry

            lax.fori_loop(0, nchunk, body, 0)

    return gather


_gather2 = _make_gather(2, 640)
_gather1w = _make_gather(1, 640)


def _pad128_mxu(table):
    eye = jnp.eye(_D, _DP, dtype=jnp.float32)
    return lax.dot_general(
        table, eye, (((1,), (0,)), ((), ())),
        precision=lax.Precision.HIGHEST,
        preferred_element_type=jnp.float32,
    )


def kernel(words, tags, lemmas, word_table, tag_table, lemma_table):
    ot, ol = _gather2(
        tags.reshape(-1), lemmas.reshape(-1),
        _pad128_mxu(tag_table), _pad128_mxu(lemma_table),
    )
    ow, = _gather1w(words.reshape(-1), _pad128_mxu(word_table))
    embed = jnp.concatenate([ow[:, :_D], ot[:, :_D], ol[:, :_D]], axis=-1)
    return embed.reshape(_B, _L, 3 * _D)
